# baseline (speedup 1.0000x reference)
import functools

import jax
import jax.numpy as jnp
from jax import lax
from jax.experimental import pallas as pl
from jax.experimental.pallas import tpu as pltpu
from jax.experimental.pallas import tpu_sc as plsc

S = 2048
H = 1024
I = 384
NS = 2
E = 8
TM = 512
NT_R = 16
CAP = NT_R * TM
NT_S = S // TM
G = NT_R + NS * NT_S
TPW = S // 32
CHUNK = 32

C_XS = 0
C_SW = 1
C_KIND = 2
C_RW = 3
C_SH = 4


def _route_body(x_ref, rw_ref, rb_ref,
                pos1_ref, pos2_ref, w1_ref, w2_ref, usage_ref, sched_ref,
                xb_ref):
    x = x_ref[...]
    xb_ref[...] = x.astype(jnp.bfloat16)
    logits = lax.dot_general(x, rw_ref[...], (((1,), (1,)), ((), ())),
                             preferred_element_type=jnp.float32)
    logits = logits + rb_ref[...]
    sig = jax.nn.sigmoid(logits)
    col = lax.broadcasted_iota(jnp.int32, (S, E), 1)
    m1 = jnp.max(sig, axis=1, keepdims=True)
    i1 = jnp.min(jnp.where(sig == m1, col, E), axis=1, keepdims=True)
    sig2 = jnp.where(col == i1, -jnp.inf, sig)
    m2 = jnp.max(sig2, axis=1, keepdims=True)
    i2 = jnp.min(jnp.where(sig2 == m2, col, E), axis=1, keepdims=True)
    denom = m1 + m2
    w1_ref[...] = m1 / denom
    w2_ref[...] = m2 / denom

    cnt = ((col == i1) | (col == i2)).astype(jnp.float32)
    r_i = lax.broadcasted_iota(jnp.int32, (S, S), 0)
    c_i = lax.broadcasted_iota(jnp.int32, (S, S), 1)
    ltri = (c_i < r_i).astype(jnp.bfloat16)
    excl = lax.dot_general(ltri, cnt.astype(jnp.bfloat16),
                           (((1,), (0,)), ((), ())),
                           preferred_element_type=jnp.float32)
    counts = jnp.sum(cnt, axis=0, keepdims=True)

    ucol = lax.broadcasted_iota(jnp.int32, (S, 128), 1)
    oh = ((ucol == i1) | (ucol == i2)).astype(jnp.float32)
    usage_ref[...] = jnp.sum(oh, axis=0, keepdims=True)

    cnt_i = counts.astype(jnp.int32)
    ptiles = (cnt_i + (TM - 1)) // TM
    e_ut = (lax.broadcasted_iota(jnp.int32, (E, E), 0)
            < lax.broadcasted_iota(jnp.int32, (E, E), 1)).astype(jnp.float32)
    base_tiles = lax.dot_general(
        ptiles.astype(jnp.float32), e_ut, (((1,), (0,)), ((), ())),
        preferred_element_type=jnp.float32).astype(jnp.int32)
    base_rows = base_tiles * TM
    total_tiles = jnp.sum(ptiles)

    b1 = jnp.sum(jnp.where(col == i1, jnp.broadcast_to(base_rows, (S, E)), 0),
                 axis=1, keepdims=True)
    b2 = jnp.sum(jnp.where(col == i2, jnp.broadcast_to(base_rows, (S, E)), 0),
                 axis=1, keepdims=True)
    x1 = jnp.sum(jnp.where(col == i1, excl, 0.0), axis=1, keepdims=True)
    x2 = jnp.sum(jnp.where(col == i2, excl, 0.0), axis=1, keepdims=True)
    pos1_ref[...] = b1 + x1.astype(jnp.int32)
    pos2_ref[...] = b2 + x2.astype(jnp.int32)

    ii = lax.broadcasted_iota(jnp.int32, (G, 1), 0)
    iie = lax.broadcasted_iota(jnp.int32, (G, E), 1)
    sel = ((lax.broadcasted_iota(jnp.int32, (G, 1), 0)
            >= jnp.broadcast_to(base_tiles, (G, E)))
           & (lax.broadcasted_iota(jnp.int32, (G, 1), 0)
              < jnp.broadcast_to(base_tiles + ptiles, (G, E))))
    w_routed = jnp.sum(jnp.where(sel, iie, 0), axis=1, keepdims=True)
    lastw = jnp.max(jnp.where(ptiles > 0,
                              lax.broadcasted_iota(jnp.int32, (1, E), 1), 0))
    is_sh = ii >= NT_R
    jj = ii - NT_R
    a_xs = jnp.minimum(ii, total_tiles - 1)
    a_xs = jnp.where(is_sh, total_tiles - 1, a_xs)
    a_sw = jnp.where(is_sh, jj % NS, 0)
    a_kind = jnp.where(is_sh, 2, jnp.where(ii < total_tiles, 1, 0))
    a_rw = jnp.where(is_sh | (ii >= total_tiles), lastw, w_routed)
    a_sh = jnp.where(is_sh, jj // NS, 0)
    sched_ref[:, C_XS:C_XS + 1] = a_xs
    sched_ref[:, C_SW:C_SW + 1] = a_sw
    sched_ref[:, C_KIND:C_KIND + 1] = a_kind
    sched_ref[:, C_RW:C_RW + 1] = a_rw
    sched_ref[:, C_SH:C_SH + 1] = a_sh
    sched_ref[:, 5:8] = jnp.zeros((G, 3), jnp.int32)


def _route(x2d, rw, rb):
    return pl.pallas_call(
        _route_body,
        grid=(1,),
        in_specs=[
            pl.BlockSpec((S, H), lambda i: (0, 0)),
            pl.BlockSpec((E, H), lambda i: (0, 0)),
            pl.BlockSpec((1, E), lambda i: (0, 0)),
        ],
        out_specs=[
            pl.BlockSpec((S, 1), lambda i: (0, 0)),
            pl.BlockSpec((S, 1), lambda i: (0, 0)),
            pl.BlockSpec((S, 1), lambda i: (0, 0)),
            pl.BlockSpec((S, 1), lambda i: (0, 0)),
            pl.BlockSpec((1, 128), lambda i: (0, 0)),
            pl.BlockSpec((G, 8), lambda i: (0, 0)),
            pl.BlockSpec((S, H), lambda i: (0, 0)),
        ],
        out_shape=[
            jax.ShapeDtypeStruct((S, 1), jnp.int32),
            jax.ShapeDtypeStruct((S, 1), jnp.int32),
            jax.ShapeDtypeStruct((S, 1), jnp.float32),
            jax.ShapeDtypeStruct((S, 1), jnp.float32),
            jax.ShapeDtypeStruct((1, 128), jnp.float32),
            jax.ShapeDtypeStruct((G, 8), jnp.int32),
            jax.ShapeDtypeStruct((S, H), jnp.bfloat16),
        ],
    )(x2d, rw, rb)


@functools.lru_cache(maxsize=None)
def _make_dispatch():
    mesh = plsc.VectorSubcoreMesh(core_axis_name="c", subcore_axis_name="s")

    @functools.partial(
        pl.kernel,
        out_type=jax.ShapeDtypeStruct((CAP, H), jnp.float32),
        mesh=mesh,
        scratch_types=[
            pltpu.VMEM((TPW, H), jnp.float32),
            pltpu.VMEM((TPW,), jnp.int32),
            pltpu.VMEM((TPW,), jnp.int32),
            pltpu.SemaphoreType.DMA,
            pltpu.SemaphoreType.DMA,
        ],
    )
    def _dispatch(x_hbm, pos1_hbm, pos2_hbm, xs_hbm, xin_v, p1_v, p2_v, s1, s2):
        wid = lax.axis_index("s") * 2 + lax.axis_index("c")
        base = wid * TPW
        pltpu.sync_copy(x_hbm.at[pl.ds(base, TPW)], xin_v)
        pltpu.sync_copy(pos1_hbm.at[pl.ds(base, TPW)], p1_v)
        pltpu.sync_copy(pos2_hbm.at[pl.ds(base, TPW)], p2_v)
        c1 = pltpu.async_copy(xin_v, xs_hbm.at[p1_v], s1)
        c2 = pltpu.async_copy(xin_v, xs_hbm.at[p2_v], s2)
        c1.wait()
        c2.wait()

    return _dispatch


def _ffn(xt, g_ref, u_ref, d_ref):
    g = g_ref[0].astype(jnp.bfloat16)
    u = u_ref[0].astype(jnp.bfloat16)
    d = d_ref[0].astype(jnp.bfloat16)
    gx = lax.dot_general(xt, g, (((1,), (1,)), ((), ())),
                         preferred_element_type=jnp.float32)
    ux = lax.dot_general(xt, u, (((1,), (1,)), ((), ())),
                         preferred_element_type=jnp.float32)
    hb = ((gx * jax.nn.sigmoid(gx)) * ux).astype(jnp.bfloat16)
    return lax.dot_general(hb, d, (((1,), (1,)), ((), ())),
                           preferred_element_type=jnp.float32)


def _gmm_body(sched_ref, xs_ref, rg_ref, ru_ref, rd_ref, rt_ref):
    i = pl.program_id(0)

    @pl.when(sched_ref[i, C_KIND] == 1)
    def _routed():
        xt = xs_ref[...].astype(jnp.bfloat16)
        rt_ref[...] = _ffn(xt, rg_ref, ru_ref, rd_ref)


def _gmm(sched, xs, rg, ru, rd):
    grid_spec = pltpu.PrefetchScalarGridSpec(
        num_scalar_prefetch=1,
        grid=(NT_R,),
        in_specs=[
            pl.BlockSpec((TM, H), lambda i, sc: (sc[i, C_XS], 0)),
            pl.BlockSpec((1, I, H), lambda i, sc: (sc[i, C_RW], 0, 0)),
            pl.BlockSpec((1, I, H), lambda i, sc: (sc[i, C_RW], 0, 0)),
            pl.BlockSpec((1, H, I), lambda i, sc: (sc[i, C_RW], 0, 0)),
        ],
        out_specs=[
            pl.BlockSpec((TM, H), lambda i, sc: (sc[i, C_XS], 0)),
        ],
    )
    return pl.pallas_call(
        _gmm_body,
        grid_spec=grid_spec,
        out_shape=[
            jax.ShapeDtypeStruct((CAP, H), jnp.float32),
        ],
    )(sched, xs, rg, ru, rd)[0]


def _shared_body(xb_ref, sg_ref, su_ref, sd_ref, sh_ref):
    e = pl.program_id(1)
    contrib = _ffn(xb_ref[...], sg_ref, su_ref, sd_ref)

    @pl.when(e == 0)
    def _():
        sh_ref[...] = contrib

    @pl.when(e != 0)
    def _():
        sh_ref[...] += contrib


def _shared_ffn(xb, sg, su, sd):
    return pl.pallas_call(
        _shared_body,
        grid=(NT_S, NS),
        in_specs=[
            pl.BlockSpec((TM, H), lambda t, e: (t, 0)),
            pl.BlockSpec((1, I, H), lambda t, e: (e, 0, 0)),
            pl.BlockSpec((1, I, H), lambda t, e: (e, 0, 0)),
            pl.BlockSpec((1, H, I), lambda t, e: (e, 0, 0)),
        ],
        out_specs=pl.BlockSpec((TM, H), lambda t, e: (t, 0)),
        out_shape=jax.ShapeDtypeStruct((S, H), jnp.float32),
    )(xb, sg, su, sd)


@functools.lru_cache(maxsize=None)
def _make_unsort():
    mesh = plsc.VectorSubcoreMesh(core_axis_name="c", subcore_axis_name="s")

    @functools.partial(
        pl.kernel,
        out_type=[
            jax.ShapeDtypeStruct((S, H), jnp.float32),
            jax.ShapeDtypeStruct((S, H), jnp.float32),
        ],
        mesh=mesh,
        scratch_types=[
            pltpu.VMEM((CHUNK, H), jnp.float32),
            pltpu.VMEM((CHUNK, H), jnp.float32),
            pltpu.VMEM((CHUNK,), jnp.int32),
            pltpu.VMEM((CHUNK,), jnp.int32),
            pltpu.VMEM((CHUNK,), jnp.int32),
            pltpu.VMEM((CHUNK,), jnp.int32),
            pltpu.SemaphoreType.DMA,
            pltpu.SemaphoreType.DMA,
            pltpu.SemaphoreType.DMA,
            pltpu.SemaphoreType.DMA,
        ],
    )
    def _unsort(rt_hbm, pos1_hbm, pos2_hbm, u1_hbm, u2_hbm,
                r1, r2, i10, i20, i11, i21, s1, s2, s3, s4):
        wid = lax.axis_index("s") * 2 + lax.axis_index("c")
        base = wid * TPW
        pltpu.sync_copy(pos1_hbm.at[pl.ds(base, CHUNK)], i10)
        pltpu.sync_copy(pos2_hbm.at[pl.ds(base, CHUNK)], i20)
        pltpu.sync_copy(pos1_hbm.at[pl.ds(base + CHUNK, CHUNK)], i11)
        pltpu.sync_copy(pos2_hbm.at[pl.ds(base + CHUNK, CHUNK)], i21)
        g1 = pltpu.async_copy(rt_hbm.at[i10], r1, s1)
        g2 = pltpu.async_copy(rt_hbm.at[i20], r2, s2)
        g1.wait()
        g2.wait()
        w1 = pltpu.async_copy(r1, u1_hbm.at[pl.ds(base, CHUNK)], s3)
        w2 = pltpu.async_copy(r2, u2_hbm.at[pl.ds(base, CHUNK)], s4)
        w1.wait()
        w2.wait()
        g1 = pltpu.async_copy(rt_hbm.at[i11], r1, s1)
        g2 = pltpu.async_copy(rt_hbm.at[i21], r2, s2)
        g1.wait()
        g2.wait()
        w1 = pltpu.async_copy(r1, u1_hbm.at[pl.ds(base + CHUNK, CHUNK)], s3)
        w2 = pltpu.async_copy(r2, u2_hbm.at[pl.ds(base + CHUNK, CHUNK)], s4)
        w1.wait()
        w2.wait()

    return _unsort


def _combine_body(sh_ref, u1_ref, u2_ref, w1_ref, w2_ref, out_ref):
    out_ref[...] = (sh_ref[...]
                    + w1_ref[...] * u1_ref[...]
                    + w2_ref[...] * u2_ref[...])


def _combine(sh, u1, u2, w1, w2):
    return pl.pallas_call(
        _combine_body,
        grid=(NT_S,),
        in_specs=[
            pl.BlockSpec((TM, H), lambda t: (t, 0)),
            pl.BlockSpec((TM, H), lambda t: (t, 0)),
            pl.BlockSpec((TM, H), lambda t: (t, 0)),
            pl.BlockSpec((TM, 1), lambda t: (t, 0)),
            pl.BlockSpec((TM, 1), lambda t: (t, 0)),
        ],
        out_specs=pl.BlockSpec((TM, H), lambda t: (t, 0)),
        out_shape=jax.ShapeDtypeStruct((S, H), jnp.float32),
    )(sh, u1, u2, w1, w2)


def kernel(x, shared_gate, shared_up, shared_down, routed_gate, routed_up,
           routed_down, router_w, router_bias):
    x2d = x.reshape(S, H)
    rb = router_bias.reshape(1, E)

    pos1, pos2, w1, w2, usage, sched, xb = _route(x2d, router_w, rb)
    p1 = pos1.reshape(S)
    p2 = pos2.reshape(S)
    xs = _make_dispatch()(x2d, p1, p2)
    sh = _shared_ffn(xb, shared_gate, shared_up, shared_down)
    rt = _gmm(sched, xs, routed_gate, routed_up, routed_down)
    u1, u2 = _make_unsort()(rt, p1, p2)
    out = _combine(sh, u1, u2, w1, w2)
    return out.reshape(x.shape), usage[0, :E]


# speedup vs baseline: 1.0201x; 1.0201x over previous
import functools

import jax
import jax.numpy as jnp
from jax import lax
from jax.experimental import pallas as pl
from jax.experimental.pallas import tpu as pltpu
from jax.experimental.pallas import tpu_sc as plsc

S = 2048
H = 1024
I = 384
NS = 2
E = 8
TM = 512
NT_R = 16
CAP = NT_R * TM
NT_S = S // TM
G = NT_R + NS * NT_S
TPW = S // 32
CHUNK = 32

C_XS = 0
C_SW = 1
C_KIND = 2
C_RW = 3
C_SH = 4


def _route_body(x_ref, rw_ref, rb_ref,
                pos1_ref, pos2_ref, w1_ref, w2_ref, usage_ref, sched_ref):
    x = x_ref[...]
    logits = lax.dot_general(x, rw_ref[...], (((1,), (1,)), ((), ())),
                             preferred_element_type=jnp.float32)
    logits = logits + rb_ref[...]
    sig = jax.nn.sigmoid(logits)
    col = lax.broadcasted_iota(jnp.int32, (S, E), 1)
    m1 = jnp.max(sig, axis=1, keepdims=True)
    i1 = jnp.min(jnp.where(sig == m1, col, E), axis=1, keepdims=True)
    sig2 = jnp.where(col == i1, -jnp.inf, sig)
    m2 = jnp.max(sig2, axis=1, keepdims=True)
    i2 = jnp.min(jnp.where(sig2 == m2, col, E), axis=1, keepdims=True)
    denom = m1 + m2
    w1_ref[...] = m1 / denom
    w2_ref[...] = m2 / denom

    cnt = ((col == i1) | (col == i2)).astype(jnp.float32)
    r_i = lax.broadcasted_iota(jnp.int32, (S, S), 0)
    c_i = lax.broadcasted_iota(jnp.int32, (S, S), 1)
    ltri = (c_i < r_i).astype(jnp.bfloat16)
    excl = lax.dot_general(ltri, cnt.astype(jnp.bfloat16),
                           (((1,), (0,)), ((), ())),
                           preferred_element_type=jnp.float32)
    counts = jnp.sum(cnt, axis=0, keepdims=True)

    ucol = lax.broadcasted_iota(jnp.int32, (S, 128), 1)
    oh = ((ucol == i1) | (ucol == i2)).astype(jnp.float32)
    usage_ref[...] = jnp.sum(oh, axis=0, keepdims=True)

    cnt_i = counts.astype(jnp.int32)
    ptiles = (cnt_i + (TM - 1)) // TM
    e_ut = (lax.broadcasted_iota(jnp.int32, (E, E), 0)
            < lax.broadcasted_iota(jnp.int32, (E, E), 1)).astype(jnp.float32)
    base_tiles = lax.dot_general(
        ptiles.astype(jnp.float32), e_ut, (((1,), (0,)), ((), ())),
        preferred_element_type=jnp.float32).astype(jnp.int32)
    base_rows = base_tiles * TM
    total_tiles = jnp.sum(ptiles)

    b1 = jnp.sum(jnp.where(col == i1, jnp.broadcast_to(base_rows, (S, E)), 0),
                 axis=1, keepdims=True)
    b2 = jnp.sum(jnp.where(col == i2, jnp.broadcast_to(base_rows, (S, E)), 0),
                 axis=1, keepdims=True)
    x1 = jnp.sum(jnp.where(col == i1, excl, 0.0), axis=1, keepdims=True)
    x2 = jnp.sum(jnp.where(col == i2, excl, 0.0), axis=1, keepdims=True)
    pos1_ref[...] = b1 + x1.astype(jnp.int32)
    pos2_ref[...] = b2 + x2.astype(jnp.int32)

    ii = lax.broadcasted_iota(jnp.int32, (G, 1), 0)
    iie = lax.broadcasted_iota(jnp.int32, (G, E), 1)
    sel = ((lax.broadcasted_iota(jnp.int32, (G, 1), 0)
            >= jnp.broadcast_to(base_tiles, (G, E)))
           & (lax.broadcasted_iota(jnp.int32, (G, 1), 0)
              < jnp.broadcast_to(base_tiles + ptiles, (G, E))))
    w_routed = jnp.sum(jnp.where(sel, iie, 0), axis=1, keepdims=True)
    lastw = jnp.max(jnp.where(ptiles > 0,
                              lax.broadcasted_iota(jnp.int32, (1, E), 1), 0))
    is_sh = ii >= NT_R
    jj = ii - NT_R
    a_xs = jnp.minimum(ii, total_tiles - 1)
    a_xs = jnp.where(is_sh, total_tiles - 1, a_xs)
    a_sw = jnp.where(is_sh, jj % NS, 0)
    a_kind = jnp.where(is_sh, 2, jnp.where(ii < total_tiles, 1, 0))
    a_rw = jnp.where(is_sh | (ii >= total_tiles), lastw, w_routed)
    a_sh = jnp.where(is_sh, jj // NS, 0)
    sched_ref[:, C_XS:C_XS + 1] = a_xs
    sched_ref[:, C_SW:C_SW + 1] = a_sw
    sched_ref[:, C_KIND:C_KIND + 1] = a_kind
    sched_ref[:, C_RW:C_RW + 1] = a_rw
    sched_ref[:, C_SH:C_SH + 1] = a_sh
    sched_ref[:, 5:8] = jnp.zeros((G, 3), jnp.int32)


def _route(x2d, rw, rb):
    return pl.pallas_call(
        _route_body,
        grid=(1,),
        in_specs=[
            pl.BlockSpec((S, H), lambda i: (0, 0)),
            pl.BlockSpec((E, H), lambda i: (0, 0)),
            pl.BlockSpec((1, E), lambda i: (0, 0)),
        ],
        out_specs=[
            pl.BlockSpec((S, 1), lambda i: (0, 0)),
            pl.BlockSpec((S, 1), lambda i: (0, 0)),
            pl.BlockSpec((S, 1), lambda i: (0, 0)),
            pl.BlockSpec((S, 1), lambda i: (0, 0)),
            pl.BlockSpec((1, 128), lambda i: (0, 0)),
            pl.BlockSpec((G, 8), lambda i: (0, 0)),
        ],
        out_shape=[
            jax.ShapeDtypeStruct((S, 1), jnp.int32),
            jax.ShapeDtypeStruct((S, 1), jnp.int32),
            jax.ShapeDtypeStruct((S, 1), jnp.float32),
            jax.ShapeDtypeStruct((S, 1), jnp.float32),
            jax.ShapeDtypeStruct((1, 128), jnp.float32),
            jax.ShapeDtypeStruct((G, 8), jnp.int32),
        ],
    )(x2d, rw, rb)


@functools.lru_cache(maxsize=None)
def _make_dispatch():
    mesh = plsc.VectorSubcoreMesh(core_axis_name="c", subcore_axis_name="s")

    @functools.partial(
        pl.kernel,
        out_type=jax.ShapeDtypeStruct((CAP, H), jnp.float32),
        mesh=mesh,
        scratch_types=[
            pltpu.VMEM((TPW, H), jnp.float32),
            pltpu.VMEM((TPW,), jnp.int32),
            pltpu.VMEM((TPW,), jnp.int32),
            pltpu.SemaphoreType.DMA,
            pltpu.SemaphoreType.DMA,
        ],
    )
    def _dispatch(x_hbm, pos1_hbm, pos2_hbm, xs_hbm, xin_v, p1_v, p2_v, s1, s2):
        wid = lax.axis_index("s") * 2 + lax.axis_index("c")
        base = wid * TPW
        pltpu.sync_copy(x_hbm.at[pl.ds(base, TPW)], xin_v)
        pltpu.sync_copy(pos1_hbm.at[pl.ds(base, TPW)], p1_v)
        pltpu.sync_copy(pos2_hbm.at[pl.ds(base, TPW)], p2_v)
        c1 = pltpu.async_copy(xin_v, xs_hbm.at[p1_v], s1)
        c2 = pltpu.async_copy(xin_v, xs_hbm.at[p2_v], s2)
        c1.wait()
        c2.wait()

    return _dispatch


def _ffn(xt, g_ref, u_ref, d_ref):
    g = g_ref[0].astype(jnp.bfloat16)
    u = u_ref[0].astype(jnp.bfloat16)
    d = d_ref[0].astype(jnp.bfloat16)
    gx = lax.dot_general(xt, g, (((1,), (1,)), ((), ())),
                         preferred_element_type=jnp.float32)
    ux = lax.dot_general(xt, u, (((1,), (1,)), ((), ())),
                         preferred_element_type=jnp.float32)
    hb = ((gx * jax.nn.sigmoid(gx)) * ux).astype(jnp.bfloat16)
    return lax.dot_general(hb, d, (((1,), (1,)), ((), ())),
                           preferred_element_type=jnp.float32)


def _gmm_body(sched_ref, xs_ref, rg_ref, ru_ref, rd_ref, rt_ref):
    i = pl.program_id(0)

    @pl.when(sched_ref[i, C_KIND] == 1)
    def _routed():
        xt = xs_ref[...].astype(jnp.bfloat16)
        rt_ref[...] = _ffn(xt, rg_ref, ru_ref, rd_ref)


def _gmm(sched, xs, rg, ru, rd):
    grid_spec = pltpu.PrefetchScalarGridSpec(
        num_scalar_prefetch=1,
        grid=(NT_R,),
        in_specs=[
            pl.BlockSpec((TM, H), lambda i, sc: (sc[i, C_XS], 0)),
            pl.BlockSpec((1, I, H), lambda i, sc: (sc[i, C_RW], 0, 0)),
            pl.BlockSpec((1, I, H), lambda i, sc: (sc[i, C_RW], 0, 0)),
            pl.BlockSpec((1, H, I), lambda i, sc: (sc[i, C_RW], 0, 0)),
        ],
        out_specs=[
            pl.BlockSpec((TM, H), lambda i, sc: (sc[i, C_XS], 0)),
        ],
    )
    return pl.pallas_call(
        _gmm_body,
        grid_spec=grid_spec,
        out_shape=[
            jax.ShapeDtypeStruct((CAP, H), jnp.float32),
        ],
    )(sched, xs, rg, ru, rd)[0]


def _shared_body(x_ref, sg_ref, su_ref, sd_ref, sh_ref):
    xt = x_ref[...].astype(jnp.bfloat16)
    sh_ref[...] = _ffn(xt, sg_ref, su_ref, sd_ref).astype(jnp.bfloat16)


def _shared_ffn_e(x2d, sg, su, sd, e):
    return pl.pallas_call(
        _shared_body,
        grid=(NT_S,),
        in_specs=[
            pl.BlockSpec((TM, H), lambda t: (t, 0)),
            pl.BlockSpec((1, I, H), lambda t: (e, 0, 0)),
            pl.BlockSpec((1, I, H), lambda t: (e, 0, 0)),
            pl.BlockSpec((1, H, I), lambda t: (e, 0, 0)),
        ],
        out_specs=pl.BlockSpec((TM, H), lambda t: (t, 0)),
        out_shape=jax.ShapeDtypeStruct((S, H), jnp.bfloat16),
    )(x2d, sg, su, sd)


@functools.lru_cache(maxsize=None)
def _make_unsort():
    mesh = plsc.VectorSubcoreMesh(core_axis_name="c", subcore_axis_name="s")

    @functools.partial(
        pl.kernel,
        out_type=[
            jax.ShapeDtypeStruct((S, H), jnp.float32),
            jax.ShapeDtypeStruct((S, H), jnp.float32),
        ],
        mesh=mesh,
        scratch_types=[
            pltpu.VMEM((CHUNK, H), jnp.float32),
            pltpu.VMEM((CHUNK, H), jnp.float32),
            pltpu.VMEM((CHUNK,), jnp.int32),
            pltpu.VMEM((CHUNK,), jnp.int32),
            pltpu.VMEM((CHUNK,), jnp.int32),
            pltpu.VMEM((CHUNK,), jnp.int32),
            pltpu.SemaphoreType.DMA,
            pltpu.SemaphoreType.DMA,
            pltpu.SemaphoreType.DMA,
            pltpu.SemaphoreType.DMA,
        ],
    )
    def _unsort(rt_hbm, pos1_hbm, pos2_hbm, u1_hbm, u2_hbm,
                r1, r2, i10, i20, i11, i21, s1, s2, s3, s4):
        wid = lax.axis_index("s") * 2 + lax.axis_index("c")
        base = wid * TPW
        pltpu.sync_copy(pos1_hbm.at[pl.ds(base, CHUNK)], i10)
        pltpu.sync_copy(pos2_hbm.at[pl.ds(base, CHUNK)], i20)
        pltpu.sync_copy(pos1_hbm.at[pl.ds(base + CHUNK, CHUNK)], i11)
        pltpu.sync_copy(pos2_hbm.at[pl.ds(base + CHUNK, CHUNK)], i21)
        g1 = pltpu.async_copy(rt_hbm.at[i10], r1, s1)
        g2 = pltpu.async_copy(rt_hbm.at[i20], r2, s2)
        g1.wait()
        g2.wait()
        w1 = pltpu.async_copy(r1, u1_hbm.at[pl.ds(base, CHUNK)], s3)
        w2 = pltpu.async_copy(r2, u2_hbm.at[pl.ds(base, CHUNK)], s4)
        w1.wait()
        w2.wait()
        g1 = pltpu.async_copy(rt_hbm.at[i11], r1, s1)
        g2 = pltpu.async_copy(rt_hbm.at[i21], r2, s2)
        g1.wait()
        g2.wait()
        w1 = pltpu.async_copy(r1, u1_hbm.at[pl.ds(base + CHUNK, CHUNK)], s3)
        w2 = pltpu.async_copy(r2, u2_hbm.at[pl.ds(base + CHUNK, CHUNK)], s4)
        w1.wait()
        w2.wait()

    return _unsort


def _combine_body(s0_ref, s1_ref, u1_ref, u2_ref, w1_ref, w2_ref, out_ref):
    out_ref[...] = (s0_ref[...].astype(jnp.float32)
                    + s1_ref[...].astype(jnp.float32)
                    + w1_ref[...] * u1_ref[...]
                    + w2_ref[...] * u2_ref[...])


def _combine(s0, s1, u1, u2, w1, w2):
    return pl.pallas_call(
        _combine_body,
        grid=(NT_S,),
        in_specs=[
            pl.BlockSpec((TM, H), lambda t: (t, 0)),
            pl.BlockSpec((TM, H), lambda t: (t, 0)),
            pl.BlockSpec((TM, H), lambda t: (t, 0)),
            pl.BlockSpec((TM, H), lambda t: (t, 0)),
            pl.BlockSpec((TM, 1), lambda t: (t, 0)),
            pl.BlockSpec((TM, 1), lambda t: (t, 0)),
        ],
        out_specs=pl.BlockSpec((TM, H), lambda t: (t, 0)),
        out_shape=jax.ShapeDtypeStruct((S, H), jnp.float32),
    )(s0, s1, u1, u2, w1, w2)


def kernel(x, shared_gate, shared_up, shared_down, routed_gate, routed_up,
           routed_down, router_w, router_bias):
    x2d = x.reshape(S, H)
    rb = router_bias.reshape(1, E)

    pos1, pos2, w1, w2, usage, sched = _route(x2d, router_w, rb)
    p1 = pos1.reshape(S)
    p2 = pos2.reshape(S)
    xs = _make_dispatch()(x2d, p1, p2)
    s0 = _shared_ffn_e(x2d, shared_gate, shared_up, shared_down, 0)
    rt = _gmm(sched, xs, routed_gate, routed_up, routed_down)
    s1 = _shared_ffn_e(x2d, shared_gate, shared_up, shared_down, 1)
    u1, u2 = _make_unsort()(rt, p1, p2)
    out = _combine(s0, s1, u1, u2, w1, w2)
    return out.reshape(x.shape), usage[0, :E]


# speedup vs baseline: 1.0575x; 1.0367x over previous
import functools

import jax
import jax.numpy as jnp
from jax import lax
from jax.experimental import pallas as pl
from jax.experimental.pallas import tpu as pltpu
from jax.experimental.pallas import tpu_sc as plsc

S = 2048
H = 1024
I = 384
NS = 2
E = 8
TM = 512
NT_R = 16
CAP = NT_R * TM
NT_S = S // TM
G = NT_R + NS * NT_S
TPW = S // 32
CHUNK = 32

C_XS = 0
C_SW = 1
C_KIND = 2
C_RW = 3
C_SH = 4


def _route_body(x_ref, rw_ref, rb_ref,
                pos1_ref, pos2_ref, w1_ref, w2_ref, usage_ref, sched_ref):
    x = x_ref[...]
    logits = lax.dot_general(x, rw_ref[...], (((1,), (1,)), ((), ())),
                             preferred_element_type=jnp.float32)
    logits = logits + rb_ref[...]
    sig = jax.nn.sigmoid(logits)
    col = lax.broadcasted_iota(jnp.int32, (S, E), 1)
    m1 = jnp.max(sig, axis=1, keepdims=True)
    i1 = jnp.min(jnp.where(sig == m1, col, E), axis=1, keepdims=True)
    sig2 = jnp.where(col == i1, -jnp.inf, sig)
    m2 = jnp.max(sig2, axis=1, keepdims=True)
    i2 = jnp.min(jnp.where(sig2 == m2, col, E), axis=1, keepdims=True)
    denom = m1 + m2
    w1_ref[...] = m1 / denom
    w2_ref[...] = m2 / denom

    cnt = ((col == i1) | (col == i2)).astype(jnp.float32)
    r_i = lax.broadcasted_iota(jnp.int32, (S, S), 0)
    c_i = lax.broadcasted_iota(jnp.int32, (S, S), 1)
    ltri = (c_i < r_i).astype(jnp.bfloat16)
    excl = lax.dot_general(ltri, cnt.astype(jnp.bfloat16),
                           (((1,), (0,)), ((), ())),
                           preferred_element_type=jnp.float32)
    counts = jnp.sum(cnt, axis=0, keepdims=True)

    ucol = lax.broadcasted_iota(jnp.int32, (S, 128), 1)
    oh = ((ucol == i1) | (ucol == i2)).astype(jnp.float32)
    usage_ref[...] = jnp.sum(oh, axis=0, keepdims=True)

    cnt_i = counts.astype(jnp.int32)
    ptiles = (cnt_i + (TM - 1)) // TM
    e_ut = (lax.broadcasted_iota(jnp.int32, (E, E), 0)
            < lax.broadcasted_iota(jnp.int32, (E, E), 1)).astype(jnp.float32)
    base_tiles = lax.dot_general(
        ptiles.astype(jnp.float32), e_ut, (((1,), (0,)), ((), ())),
        preferred_element_type=jnp.float32).astype(jnp.int32)
    base_rows = base_tiles * TM
    total_tiles = jnp.sum(ptiles)

    b1 = jnp.sum(jnp.where(col == i1, jnp.broadcast_to(base_rows, (S, E)), 0),
                 axis=1, keepdims=True)
    b2 = jnp.sum(jnp.where(col == i2, jnp.broadcast_to(base_rows, (S, E)), 0),
                 axis=1, keepdims=True)
    x1 = jnp.sum(jnp.where(col == i1, excl, 0.0), axis=1, keepdims=True)
    x2 = jnp.sum(jnp.where(col == i2, excl, 0.0), axis=1, keepdims=True)
    pos1_ref[...] = b1 + x1.astype(jnp.int32)
    pos2_ref[...] = b2 + x2.astype(jnp.int32)

    ii = lax.broadcasted_iota(jnp.int32, (G, 1), 0)
    iie = lax.broadcasted_iota(jnp.int32, (G, E), 1)
    sel = ((lax.broadcasted_iota(jnp.int32, (G, 1), 0)
            >= jnp.broadcast_to(base_tiles, (G, E)))
           & (lax.broadcasted_iota(jnp.int32, (G, 1), 0)
              < jnp.broadcast_to(base_tiles + ptiles, (G, E))))
    w_routed = jnp.sum(jnp.where(sel, iie, 0), axis=1, keepdims=True)
    lastw = jnp.max(jnp.where(ptiles > 0,
                              lax.broadcasted_iota(jnp.int32, (1, E), 1), 0))
    is_sh = ii >= NT_R
    jj = ii - NT_R
    a_xs = jnp.minimum(ii, total_tiles - 1)
    a_xs = jnp.where(is_sh, total_tiles - 1, a_xs)
    a_sw = jnp.where(is_sh, jj % NS, 0)
    a_kind = jnp.where(is_sh, 2, jnp.where(ii < total_tiles, 1, 0))
    a_rw = jnp.where(is_sh | (ii >= total_tiles), lastw, w_routed)
    a_sh = jnp.where(is_sh, jj // NS, 0)
    sched_ref[:, C_XS:C_XS + 1] = a_xs
    sched_ref[:, C_SW:C_SW + 1] = a_sw
    sched_ref[:, C_KIND:C_KIND + 1] = a_kind
    sched_ref[:, C_RW:C_RW + 1] = a_rw
    sched_ref[:, C_SH:C_SH + 1] = a_sh
    sched_ref[:, 5:8] = jnp.zeros((G, 3), jnp.int32)


def _route(x2d, rw, rb):
    return pl.pallas_call(
        _route_body,
        grid=(1,),
        in_specs=[
            pl.BlockSpec((S, H), lambda i: (0, 0)),
            pl.BlockSpec((E, H), lambda i: (0, 0)),
            pl.BlockSpec((1, E), lambda i: (0, 0)),
        ],
        out_specs=[
            pl.BlockSpec((S, 1), lambda i: (0, 0)),
            pl.BlockSpec((S, 1), lambda i: (0, 0)),
            pl.BlockSpec((S, 1), lambda i: (0, 0)),
            pl.BlockSpec((S, 1), lambda i: (0, 0)),
            pl.BlockSpec((1, 128), lambda i: (0, 0)),
            pl.BlockSpec((G, 8), lambda i: (0, 0)),
        ],
        out_shape=[
            jax.ShapeDtypeStruct((S, 1), jnp.int32),
            jax.ShapeDtypeStruct((S, 1), jnp.int32),
            jax.ShapeDtypeStruct((S, 1), jnp.float32),
            jax.ShapeDtypeStruct((S, 1), jnp.float32),
            jax.ShapeDtypeStruct((1, 128), jnp.float32),
            jax.ShapeDtypeStruct((G, 8), jnp.int32),
        ],
    )(x2d, rw, rb)


@functools.lru_cache(maxsize=None)
def _make_dispatch():
    mesh = plsc.VectorSubcoreMesh(core_axis_name="c", subcore_axis_name="s")

    @functools.partial(
        pl.kernel,
        out_type=jax.ShapeDtypeStruct((CAP, H), jnp.float32),
        mesh=mesh,
        scratch_types=[
            pltpu.VMEM((TPW, H), jnp.float32),
            pltpu.VMEM((TPW,), jnp.int32),
            pltpu.VMEM((TPW,), jnp.int32),
            pltpu.SemaphoreType.DMA,
            pltpu.SemaphoreType.DMA,
        ],
    )
    def _dispatch(x_hbm, pos1_hbm, pos2_hbm, xs_hbm, xin_v, p1_v, p2_v, s1, s2):
        wid = lax.axis_index("s") * 2 + lax.axis_index("c")
        base = wid * TPW
        pltpu.sync_copy(x_hbm.at[pl.ds(base, TPW)], xin_v)
        pltpu.sync_copy(pos1_hbm.at[pl.ds(base, TPW)], p1_v)
        pltpu.sync_copy(pos2_hbm.at[pl.ds(base, TPW)], p2_v)
        c1 = pltpu.async_copy(xin_v, xs_hbm.at[p1_v], s1)
        c2 = pltpu.async_copy(xin_v, xs_hbm.at[p2_v], s2)
        c1.wait()
        c2.wait()

    return _dispatch


def _ffn(xt, g_ref, u_ref, d_ref):
    g = g_ref[0].astype(jnp.bfloat16)
    u = u_ref[0].astype(jnp.bfloat16)
    d = d_ref[0].astype(jnp.bfloat16)
    gx = lax.dot_general(xt, g, (((1,), (1,)), ((), ())),
                         preferred_element_type=jnp.float32)
    ux = lax.dot_general(xt, u, (((1,), (1,)), ((), ())),
                         preferred_element_type=jnp.float32)
    hb = ((gx * jax.nn.sigmoid(gx)) * ux).astype(jnp.bfloat16)
    return lax.dot_general(hb, d, (((1,), (1,)), ((), ())),
                           preferred_element_type=jnp.float32)


def _gmm_body(sched_ref, xs_ref, rg_ref, ru_ref, rd_ref, rt_ref):
    i = pl.program_id(0)

    @pl.when(sched_ref[i, C_KIND] == 1)
    def _routed():
        xt = xs_ref[...].astype(jnp.bfloat16)
        rt_ref[...] = _ffn(xt, rg_ref, ru_ref, rd_ref)


def _gmm(sched, xs, rg, ru, rd):
    grid_spec = pltpu.PrefetchScalarGridSpec(
        num_scalar_prefetch=1,
        grid=(NT_R,),
        in_specs=[
            pl.BlockSpec((TM, H), lambda i, sc: (sc[i, C_XS], 0)),
            pl.BlockSpec((1, I, H), lambda i, sc: (sc[i, C_RW], 0, 0)),
            pl.BlockSpec((1, I, H), lambda i, sc: (sc[i, C_RW], 0, 0)),
            pl.BlockSpec((1, H, I), lambda i, sc: (sc[i, C_RW], 0, 0)),
        ],
        out_specs=[
            pl.BlockSpec((TM, H), lambda i, sc: (sc[i, C_XS], 0)),
        ],
    )
    return pl.pallas_call(
        _gmm_body,
        grid_spec=grid_spec,
        out_shape=[
            jax.ShapeDtypeStruct((CAP, H), jnp.float32),
        ],
    )(sched, xs, rg, ru, rd)[0]


def _shared_body(x_ref, sg_ref, su_ref, sd_ref, sh_ref, acc_ref):
    e = pl.program_id(0)
    t = pl.program_id(1)
    xt = x_ref[...].astype(jnp.bfloat16)
    contrib = _ffn(xt, sg_ref, su_ref, sd_ref)

    @pl.when(e == 0)
    def _():
        acc_ref[pl.ds(t * TM, TM), :] = contrib

    @pl.when(e == NS - 1)
    def _():
        sh_ref[...] = (acc_ref[pl.ds(t * TM, TM), :]
                       + contrib).astype(jnp.bfloat16)


def _shared_ffn(x2d, sg, su, sd):
    return pl.pallas_call(
        _shared_body,
        grid=(NS, NT_S),
        in_specs=[
            pl.BlockSpec((TM, H), lambda e, t: (t, 0)),
            pl.BlockSpec((1, I, H), lambda e, t: (e, 0, 0)),
            pl.BlockSpec((1, I, H), lambda e, t: (e, 0, 0)),
            pl.BlockSpec((1, H, I), lambda e, t: (e, 0, 0)),
        ],
        out_specs=pl.BlockSpec(
            (TM, H), lambda e, t: (jnp.where(e == NS - 1, t, 0), 0)),
        out_shape=jax.ShapeDtypeStruct((S, H), jnp.bfloat16),
        scratch_shapes=[pltpu.VMEM((S, H), jnp.float32)],
    )(x2d, sg, su, sd)


@functools.lru_cache(maxsize=None)
def _make_unsort():
    mesh = plsc.VectorSubcoreMesh(core_axis_name="c", subcore_axis_name="s")

    @functools.partial(
        pl.kernel,
        out_type=[
            jax.ShapeDtypeStruct((S, H), jnp.float32),
            jax.ShapeDtypeStruct((S, H), jnp.float32),
        ],
        mesh=mesh,
        scratch_types=[
            pltpu.VMEM((CHUNK, H), jnp.float32),
            pltpu.VMEM((CHUNK, H), jnp.float32),
            pltpu.VMEM((CHUNK,), jnp.int32),
            pltpu.VMEM((CHUNK,), jnp.int32),
            pltpu.VMEM((CHUNK,), jnp.int32),
            pltpu.VMEM((CHUNK,), jnp.int32),
            pltpu.SemaphoreType.DMA,
            pltpu.SemaphoreType.DMA,
            pltpu.SemaphoreType.DMA,
            pltpu.SemaphoreType.DMA,
        ],
    )
    def _unsort(rt_hbm, pos1_hbm, pos2_hbm, u1_hbm, u2_hbm,
                r1, r2, i10, i20, i11, i21, s1, s2, s3, s4):
        wid = lax.axis_index("s") * 2 + lax.axis_index("c")
        base = wid * TPW
        pltpu.sync_copy(pos1_hbm.at[pl.ds(base, CHUNK)], i10)
        pltpu.sync_copy(pos2_hbm.at[pl.ds(base, CHUNK)], i20)
        pltpu.sync_copy(pos1_hbm.at[pl.ds(base + CHUNK, CHUNK)], i11)
        pltpu.sync_copy(pos2_hbm.at[pl.ds(base + CHUNK, CHUNK)], i21)
        g1 = pltpu.async_copy(rt_hbm.at[i10], r1, s1)
        g2 = pltpu.async_copy(rt_hbm.at[i20], r2, s2)
        g1.wait()
        g2.wait()
        w1 = pltpu.async_copy(r1, u1_hbm.at[pl.ds(base, CHUNK)], s3)
        w2 = pltpu.async_copy(r2, u2_hbm.at[pl.ds(base, CHUNK)], s4)
        w1.wait()
        w2.wait()
        g1 = pltpu.async_copy(rt_hbm.at[i11], r1, s1)
        g2 = pltpu.async_copy(rt_hbm.at[i21], r2, s2)
        g1.wait()
        g2.wait()
        w1 = pltpu.async_copy(r1, u1_hbm.at[pl.ds(base + CHUNK, CHUNK)], s3)
        w2 = pltpu.async_copy(r2, u2_hbm.at[pl.ds(base + CHUNK, CHUNK)], s4)
        w1.wait()
        w2.wait()

    return _unsort


def _combine_body(sh_ref, u1_ref, u2_ref, w1_ref, w2_ref, out_ref):
    out_ref[...] = (sh_ref[...].astype(jnp.float32)
                    + w1_ref[...] * u1_ref[...]
                    + w2_ref[...] * u2_ref[...])


def _combine(sh, u1, u2, w1, w2):
    return pl.pallas_call(
        _combine_body,
        grid=(NT_S,),
        in_specs=[
            pl.BlockSpec((TM, H), lambda t: (t, 0)),
            pl.BlockSpec((TM, H), lambda t: (t, 0)),
            pl.BlockSpec((TM, H), lambda t: (t, 0)),
            pl.BlockSpec((TM, 1), lambda t: (t, 0)),
            pl.BlockSpec((TM, 1), lambda t: (t, 0)),
        ],
        out_specs=pl.BlockSpec((TM, H), lambda t: (t, 0)),
        out_shape=jax.ShapeDtypeStruct((S, H), jnp.float32),
    )(sh, u1, u2, w1, w2)


def kernel(x, shared_gate, shared_up, shared_down, routed_gate, routed_up,
           routed_down, router_w, router_bias):
    x2d = x.reshape(S, H)
    rb = router_bias.reshape(1, E)

    pos1, pos2, w1, w2, usage, sched = _route(x2d, router_w, rb)
    p1 = pos1.reshape(S)
    p2 = pos2.reshape(S)
    xs = _make_dispatch()(x2d, p1, p2)
    sh = _shared_ffn(x2d, shared_gate, shared_up, shared_down)
    rt = _gmm(sched, xs, routed_gate, routed_up, routed_down)
    u1, u2 = _make_unsort()(rt, p1, p2)
    out = _combine(sh, u1, u2, w1, w2)
    return out.reshape(x.shape), usage[0, :E]
